# baseline (speedup 1.0000x reference)
import functools

import jax
import jax.numpy as jnp
from jax import lax
from jax.experimental import pallas as pl
from jax.experimental.pallas import tpu as pltpu
from jax.experimental.pallas import tpu_sc as plsc

NC, NS, L = 2, 16, 16
NW = NC * NS
NEG = 0.2
BM = 2048


def _prologue_body(x_ref, w_ref, a2_ref, ha_ref, hb_ref, as_ref, ad_ref):
    h = jnp.dot(x_ref[...], w_ref[...], preferred_element_type=jnp.float32)
    hf = h.shape[1] // 2
    ha_ref[...] = h[:, :hf]
    hb_ref[...] = h[:, hf:]
    ap = jnp.dot(h, a2_ref[...], preferred_element_type=jnp.float32)
    as_ref[...] = ap[:, 0]
    ad_ref[...] = ap[:, 1]


def _tc_prologue(xp, W, A2):
    NP, Fin = xp.shape
    H = W.shape[1]
    return pl.pallas_call(
        _prologue_body,
        grid=(NP // BM,),
        in_specs=[pl.BlockSpec((BM, Fin), lambda i: (i, 0)),
                  pl.BlockSpec((Fin, H), lambda i: (0, 0)),
                  pl.BlockSpec((H, 2), lambda i: (0, 0))],
        out_specs=[pl.BlockSpec((BM, H // 2), lambda i: (i, 0)),
                   pl.BlockSpec((BM, H // 2), lambda i: (i, 0)),
                   pl.BlockSpec((BM,), lambda i: (i,)),
                   pl.BlockSpec((BM,), lambda i: (i,))],
        out_shape=[jax.ShapeDtypeStruct((NP, H // 2), jnp.float32),
                   jax.ShapeDtypeStruct((NP, H // 2), jnp.float32),
                   jax.ShapeDtypeStruct((NP,), jnp.float32),
                   jax.ShapeDtypeStruct((NP,), jnp.float32)],
    )(xp, W, A2)


def _make_mid_body(n_valid):
    def _mid_body(p_ref, d_ref, b_ref, w_ref, a2_ref,
                  h_ref, as_ref, ad_ref):
        agg = jnp.concatenate([p_ref[0], p_ref[1]], axis=1)
        rden = (1.0 / (d_ref[0] + 1e-30)).reshape(-1, 1)
        z = jnp.maximum(agg * rden + b_ref[...], 0.0)
        rows = pl.program_id(0) * BM + lax.broadcasted_iota(
            jnp.int32, (BM, 1), 0)
        z = jnp.where(rows < n_valid, z, 0.0)
        h = jnp.dot(z, w_ref[...], preferred_element_type=jnp.float32)
        h_ref[...] = h
        ap = jnp.dot(h, a2_ref[...], preferred_element_type=jnp.float32)
        as_ref[...] = ap[:, 0]
        ad_ref[...] = ap[:, 1]
    return _mid_body


def _tc_mid(parts, dens, b, W, A2, n_valid):
    NP, Fh = parts.shape[1], parts.shape[2]
    H = 2 * Fh
    C = W.shape[1]
    return pl.pallas_call(
        _make_mid_body(n_valid),
        grid=(NP // BM,),
        in_specs=[pl.BlockSpec((2, BM, Fh), lambda i: (0, i, 0)),
                  pl.BlockSpec((2, BM), lambda i: (0, i)),
                  pl.BlockSpec((1, H), lambda i: (0, 0)),
                  pl.BlockSpec((H, C), lambda i: (0, 0)),
                  pl.BlockSpec((C, 2), lambda i: (0, 0))],
        out_specs=[pl.BlockSpec((BM, C), lambda i: (i, 0)),
                   pl.BlockSpec((BM,), lambda i: (i,)),
                   pl.BlockSpec((BM,), lambda i: (i,))],
        out_shape=[jax.ShapeDtypeStruct((NP, C), jnp.float32),
                   jax.ShapeDtypeStruct((NP,), jnp.float32),
                   jax.ShapeDtypeStruct((NP,), jnp.float32)],
    )(parts, dens, b, W, A2)


def _final_body(p_ref, d_ref, b_ref, o_ref):
    rden = (1.0 / (d_ref[0] + d_ref[1] + 1e-30)).reshape(-1, 1)
    o_ref[...] = (p_ref[0] + p_ref[1]) * rden + b_ref[...]


def _tc_final(parts, dens, b):
    NP, C = parts.shape[1], parts.shape[2]
    return pl.pallas_call(
        _final_body,
        grid=(NP // BM,),
        in_specs=[pl.BlockSpec((2, BM, C), lambda i: (0, i, 0)),
                  pl.BlockSpec((2, BM), lambda i: (0, i)),
                  pl.BlockSpec((1, C), lambda i: (0, 0))],
        out_specs=pl.BlockSpec((BM, C), lambda i: (i, 0)),
        out_shape=jax.ShapeDtypeStruct((NP, C), jnp.float32),
    )(parts, dens, b)


def _make_sc_layer(NP, Fh, CH, bpt):
    nS = NP // NS
    mesh = plsc.VectorSubcoreMesh(core_axis_name="c", subcore_axis_name="s")

    @functools.partial(
        pl.kernel,
        out_type=[jax.ShapeDtypeStruct((NC, NP, Fh), jnp.float32),
                  jax.ShapeDtypeStruct((NC, NP), jnp.float32)],
        mesh=mesh,
        compiler_params=pltpu.CompilerParams(
            needs_layout_passes=False,
            use_tc_tiling_on_sc=False),
        scratch_types=[
            pltpu.VMEM_SHARED((NP,), jnp.float32),
            pltpu.VMEM_SHARED((NP, Fh), jnp.float32),
            pltpu.VMEM((NP,), jnp.float32),
            pltpu.VMEM((NP,), jnp.float32),
            pltpu.VMEM((CH, 128), jnp.int32),
            pltpu.VMEM((CH, 128), jnp.int32),
            pltpu.VMEM((3, 128), jnp.float32),
            pltpu.VMEM((3, 128, Fh), jnp.float32),
        ] + [pltpu.SemaphoreType.DMA] * 9,
    )
    def sc_layer(hA, hB, asrc_hbm, adst_hbm, srcI, dstI, parts, dens,
                 den_sh, out_sh, asrc_v, adst_v,
                 srcP, dstP, wbuf, rowbuf,
                 g0, g1, g2, s0, s1, s2, d0, d1, d2):
        gsems = (g0, g1, g2)
        ssems = (s0, s1, s2)
        dsems = (d0, d1, d2)
        cid = lax.axis_index("c")
        sid = lax.axis_index("s")
        wid = cid * NS + sid
        base = sid * nS

        pltpu.sync_copy(asrc_hbm, asrc_v)
        pltpu.sync_copy(adst_hbm, adst_v)

        def zrow(i, _):
            for q in range(Fh // L):
                rowbuf[0, i, pl.ds(q * L, L)] = jnp.zeros((L,), jnp.float32)
            return 0
        lax.fori_loop(0, 128, zrow, 0)
        for q in range(128 // L):
            wbuf[0, pl.ds(q * L, L)] = jnp.zeros((L,), jnp.float32)
        for k in range(nS // 128):
            pltpu.sync_copy(rowbuf.at[0],
                            out_sh.at[pl.ds(base + k * 128, 128)])
            pltpu.sync_copy(wbuf.at[0],
                            den_sh.at[pl.ds(base + k * 128, 128)])
        plsc.subcore_barrier()

        def mx(i, m):
            return jnp.maximum(m, asrc_v[pl.ds(i * L, L)])
        m16 = lax.fori_loop(0, NP // L, mx,
                            jnp.full((L,), -3.0e38, jnp.float32))
        amax = m16[0]
        for i in range(1, L):
            amax = jnp.maximum(amax, m16[i])

        def gather_rows(g, slot, sem):
            @pl.when(cid == 0)
            def _():
                pltpu.async_copy(hA.at[srcP.at[g]], rowbuf.at[slot], sem)

            @pl.when(cid == 1)
            def _():
                pltpu.async_copy(hB.at[srcP.at[g]], rowbuf.at[slot], sem)

        def weights(g, b):
            for q in range(128 // L):
                sv = srcP[g, pl.ds(q * L, L)]
                dv = dstP[g, pl.ds(q * L, L)]
                a_s = plsc.load_gather(asrc_v, [sv])
                a_d = plsc.load_gather(adst_v, [dv])
                al = a_s + a_d
                al = jnp.maximum(al, NEG * al)
                sh = a_d + amax
                sh = jnp.maximum(sh, NEG * sh)
                wbuf[b, pl.ds(q * L, L)] = jnp.exp(al - sh)

        def scale_rows(b):
            def rgrp(t, _):
                wv = wbuf[b, pl.ds(t * L, L)]
                for rr in range(L):
                    row = t * L + rr
                    ws = wv[rr]
                    for q in range(Fh // L):
                        rowbuf[b, row, pl.ds(q * L, L)] = (
                            rowbuf[b, row, pl.ds(q * L, L)] * ws)
                return 0
            lax.fori_loop(0, 128 // L, rgrp, 0)

        def sweep_block(blk):
            pltpu.sync_copy(srcI.at[blk], srcP)
            pltpu.sync_copy(dstI.at[blk], dstP)
            gather_rows(0, 0, gsems[0])

            def piped(g2, _):
                for b in range(3):
                    g = g2 * 3 + b
                    nb = (b + 1) % 3
                    @pl.when(g2 >= 1)
                    def _():
                        pltpu.make_async_copy(
                            wbuf.at[b], den_sh.at[dstP.at[g]],
                            dsems[b]).wait()
                    weights(g, b)
                    pltpu.async_copy(wbuf.at[b], den_sh.at[dstP.at[g]],
                                     dsems[b], add=True)

                    def wait_row_scatter():
                        pltpu.make_async_copy(
                            rowbuf.at[nb], out_sh.at[dstP.at[g]],
                            ssems[nb]).wait()
                    if b < 2:
                        @pl.when(g2 >= 1)
                        def _():
                            wait_row_scatter()
                        gather_rows(g + 1, nb, gsems[nb])
                    else:
                        @pl.when(g2 < CH // 3 - 1)
                        def _():
                            wait_row_scatter()
                            gather_rows(g + 1, nb, gsems[nb])

                    pltpu.make_async_copy(hA.at[srcP.at[g]],
                                          rowbuf.at[b], gsems[b]).wait()
                    scale_rows(b)
                    pltpu.async_copy(rowbuf.at[b], out_sh.at[dstP.at[g]],
                                     ssems[b], add=True)
                return 0
            lax.fori_loop(0, CH // 3, piped, 0)

            for b in range(3):
                pltpu.make_async_copy(rowbuf.at[b],
                                      out_sh.at[dstP.at[0]],
                                      ssems[b]).wait()
                pltpu.make_async_copy(wbuf.at[b],
                                      den_sh.at[dstP.at[0]],
                                      dsems[b]).wait()

        if bpt == 1:
            sweep_block(wid)
        else:
            for bb in range(bpt):
                sweep_block(bpt * sid + bb)
        plsc.subcore_barrier()

        pltpu.sync_copy(out_sh.at[pl.ds(base, nS)],
                        parts.at[cid, pl.ds(base, nS)])
        pltpu.sync_copy(den_sh.at[pl.ds(base, nS)],
                        dens.at[cid, pl.ds(base, nS)])

    return sc_layer


def kernel(x, edge_index, W1, att_src1, att_dst1, b1,
           W2, att_src2, att_dst2, b2):
    N, Fin = x.shape
    E = edge_index.shape[1]
    H = W1.shape[1]
    C = W2.shape[1]

    NP = (N // (NS * 128) + 1) * (NS * 128)
    Et = E + N
    EP = -(-Et // (NW * 384)) * (NW * 384)
    CH = EP // (NW * 128)
    npad = EP - Et

    ei = edge_index.astype(jnp.int32)
    loops = jnp.arange(N, dtype=jnp.int32)
    pad_src = jnp.arange(npad, dtype=jnp.int32) % N
    pad_dst = N + jnp.arange(npad, dtype=jnp.int32) % (NP - N)
    src = jnp.concatenate([ei[0], loops, pad_src]).reshape(NW, CH, 128)
    dst = jnp.concatenate([ei[1], loops, pad_dst]).reshape(NW, CH, 128)

    xp = jnp.zeros((NP, Fin), jnp.float32).at[:N].set(x)
    A21 = jnp.stack([att_src1, att_dst1], axis=1)
    A22 = jnp.stack([att_src2, att_dst2], axis=1)

    hA, hB, as1, ad1 = _tc_prologue(xp, W1, A21)
    sc1 = _make_sc_layer(NP, H // 2, CH, bpt=2)
    parts1, dens1 = sc1(hA, hB, as1, ad1, src, dst)

    h2, as2, ad2 = _tc_mid(parts1, dens1, b1.reshape(1, H), W2, A22, N)
    sc2 = _make_sc_layer(NP, C, CH, bpt=1)
    parts2, dens2 = sc2(h2, h2, as2, ad2, src, dst)

    out = _tc_final(parts2, dens2, b2.reshape(1, C))
    return out[:N], edge_index


# speedup vs baseline: 1.4044x; 1.4044x over previous
import functools

import jax
import jax.numpy as jnp
from jax import lax
from jax.experimental import pallas as pl
from jax.experimental.pallas import tpu as pltpu
from jax.experimental.pallas import tpu_sc as plsc

NC, NS, L = 2, 16, 16
NW = NC * NS
NEG = 0.2
BM = 2048


def _prologue_body(x_ref, w_ref, a2_ref, h2_ref, as_ref, ad_ref):
    j = pl.program_id(0)
    h = jnp.dot(x_ref[...], w_ref[...], preferred_element_type=jnp.float32)
    hf = h.shape[1] // 2
    h2_ref[...] = jnp.where(j == 0, h[:, :hf], h[:, hf:])
    ap = jnp.dot(h, a2_ref[...], preferred_element_type=jnp.float32)
    as_ref[...] = ap[:, 0]
    ad_ref[...] = ap[:, 1]


def _tc_prologue(xp, W, A2):
    NP, Fin = xp.shape
    H = W.shape[1]
    nb = NP // BM
    return pl.pallas_call(
        _prologue_body,
        grid=(2, nb),
        in_specs=[pl.BlockSpec((BM, Fin), lambda j, i: (i, 0)),
                  pl.BlockSpec((Fin, H), lambda j, i: (0, 0)),
                  pl.BlockSpec((H, 2), lambda j, i: (0, 0))],
        out_specs=[pl.BlockSpec((BM, H // 2), lambda j, i: (j * nb + i, 0)),
                   pl.BlockSpec((BM,), lambda j, i: (i,)),
                   pl.BlockSpec((BM,), lambda j, i: (i,))],
        out_shape=[jax.ShapeDtypeStruct((2 * NP, H // 2), jnp.float32),
                   jax.ShapeDtypeStruct((NP,), jnp.float32),
                   jax.ShapeDtypeStruct((NP,), jnp.float32)],
    )(xp, W, A2)


def _make_mid_body(n_valid):
    def _mid_body(p_ref, d_ref, b_ref, w_ref, a2_ref,
                  h_ref, as_ref, ad_ref):
        agg = jnp.concatenate([p_ref[0], p_ref[1]], axis=1)
        rden = (1.0 / (d_ref[0] + 1e-30)).reshape(-1, 1)
        z = jnp.maximum(agg * rden + b_ref[...], 0.0)
        rows = pl.program_id(0) * BM + lax.broadcasted_iota(
            jnp.int32, (BM, 1), 0)
        z = jnp.where(rows < n_valid, z, 0.0)
        h = jnp.dot(z, w_ref[...], preferred_element_type=jnp.float32)
        h_ref[...] = h
        ap = jnp.dot(h, a2_ref[...], preferred_element_type=jnp.float32)
        as_ref[...] = ap[:, 0]
        ad_ref[...] = ap[:, 1]
    return _mid_body


def _tc_mid(parts, dens, b, W, A2, n_valid):
    NP, Fh = parts.shape[1], parts.shape[2]
    H = 2 * Fh
    C = W.shape[1]
    return pl.pallas_call(
        _make_mid_body(n_valid),
        grid=(NP // BM,),
        in_specs=[pl.BlockSpec((2, BM, Fh), lambda i: (0, i, 0)),
                  pl.BlockSpec((2, BM), lambda i: (0, i)),
                  pl.BlockSpec((1, H), lambda i: (0, 0)),
                  pl.BlockSpec((H, C), lambda i: (0, 0)),
                  pl.BlockSpec((C, 2), lambda i: (0, 0))],
        out_specs=[pl.BlockSpec((BM, C), lambda i: (i, 0)),
                   pl.BlockSpec((BM,), lambda i: (i,)),
                   pl.BlockSpec((BM,), lambda i: (i,))],
        out_shape=[jax.ShapeDtypeStruct((NP, C), jnp.float32),
                   jax.ShapeDtypeStruct((NP,), jnp.float32),
                   jax.ShapeDtypeStruct((NP,), jnp.float32)],
    )(parts, dens, b, W, A2)


def _final_body(p_ref, d_ref, b_ref, o_ref):
    rden = (1.0 / (d_ref[0] + d_ref[1] + 1e-30)).reshape(-1, 1)
    o_ref[...] = (p_ref[0] + p_ref[1]) * rden + b_ref[...]


def _tc_final(parts, dens, b):
    NP, C = parts.shape[1], parts.shape[2]
    return pl.pallas_call(
        _final_body,
        grid=(NP // BM,),
        in_specs=[pl.BlockSpec((2, BM, C), lambda i: (0, i, 0)),
                  pl.BlockSpec((2, BM), lambda i: (0, i)),
                  pl.BlockSpec((1, C), lambda i: (0, 0))],
        out_specs=pl.BlockSpec((BM, C), lambda i: (i, 0)),
        out_shape=jax.ShapeDtypeStruct((NP, C), jnp.float32),
    )(parts, dens, b)


def _make_sc_layer(NP, Fh, CH, bpt, off):
    nS = NP // NS
    mesh = plsc.VectorSubcoreMesh(core_axis_name="c", subcore_axis_name="s")

    @functools.partial(
        pl.kernel,
        out_type=[jax.ShapeDtypeStruct((NC, NP, Fh), jnp.float32),
                  jax.ShapeDtypeStruct((NC, NP), jnp.float32)],
        mesh=mesh,
        compiler_params=pltpu.CompilerParams(
            needs_layout_passes=False,
            use_tc_tiling_on_sc=False),
        scratch_types=[
            pltpu.VMEM_SHARED((NP,), jnp.float32),
            pltpu.VMEM_SHARED((NP, Fh), jnp.float32),
            pltpu.VMEM((NP,), jnp.float32),
            pltpu.VMEM((NP,), jnp.float32),
            pltpu.VMEM((CH, 128), jnp.int32),
            pltpu.VMEM((CH, 128), jnp.int32),
            pltpu.VMEM((3, 128), jnp.float32),
            pltpu.VMEM((3, 128, Fh), jnp.float32),
        ] + [pltpu.SemaphoreType.DMA] * 9,
    )
    def sc_layer(h2d, asrc_hbm, adst_hbm, srcI, dstI, parts, dens,
                 den_sh, out_sh, asrc_v, adst_v,
                 srcP, dstP, wbuf, rowbuf,
                 g0, g1, g2, s0, s1, s2, d0, d1, d2):
        gsems = (g0, g1, g2)
        ssems = (s0, s1, s2)
        dsems = (d0, d1, d2)
        cid = lax.axis_index("c")
        sid = lax.axis_index("s")
        wid = cid * NS + sid
        base = sid * nS
        coff = cid * off

        pltpu.sync_copy(asrc_hbm, asrc_v)
        pltpu.sync_copy(adst_hbm, adst_v)

        def zrow(i, _):
            for q in range(Fh // L):
                rowbuf[0, i, pl.ds(q * L, L)] = jnp.zeros((L,), jnp.float32)
            return 0
        lax.fori_loop(0, 128, zrow, 0)
        for q in range(128 // L):
            wbuf[0, pl.ds(q * L, L)] = jnp.zeros((L,), jnp.float32)
        for k in range(nS // 128):
            pltpu.sync_copy(rowbuf.at[0],
                            out_sh.at[pl.ds(base + k * 128, 128)])
            pltpu.sync_copy(wbuf.at[0],
                            den_sh.at[pl.ds(base + k * 128, 128)])
        plsc.subcore_barrier()

        def mx(i, m):
            return jnp.maximum(m, asrc_v[pl.ds(i * L, L)])
        m16 = lax.fori_loop(0, NP // L, mx,
                            jnp.full((L,), -3.0e38, jnp.float32))
        amax = m16[0]
        for i in range(1, L):
            amax = jnp.maximum(amax, m16[i])

        def gather_rows(g, slot, sem):
            pltpu.async_copy(h2d.at[srcP.at[g]], rowbuf.at[slot], sem)

        def weights(g, b):
            for q in range(128 // L):
                sv = srcP[g, pl.ds(q * L, L)] - coff
                dv = dstP[g, pl.ds(q * L, L)]
                a_s = plsc.load_gather(asrc_v, [sv])
                a_d = plsc.load_gather(adst_v, [dv])
                al = a_s + a_d
                al = jnp.maximum(al, NEG * al)
                sh = a_d + amax
                sh = jnp.maximum(sh, NEG * sh)
                wbuf[b, pl.ds(q * L, L)] = jnp.exp(al - sh)

        def scale_rows(b):
            def rgrp(t, _):
                wv = wbuf[b, pl.ds(t * L, L)]
                for rr in range(L):
                    row = t * L + rr
                    ws = wv[rr]
                    for q in range(Fh // L):
                        rowbuf[b, row, pl.ds(q * L, L)] = (
                            rowbuf[b, row, pl.ds(q * L, L)] * ws)
                return 0
            lax.fori_loop(0, 128 // L, rgrp, 0)

        def sweep_block(blk):
            pltpu.sync_copy(srcI.at[blk], srcP)
            pltpu.sync_copy(dstI.at[blk], dstP)
            if off:
                def addoff(g, _):
                    for q in range(128 // L):
                        srcP[g, pl.ds(q * L, L)] = (
                            srcP[g, pl.ds(q * L, L)] + coff)
                    return 0
                lax.fori_loop(0, CH, addoff, 0)
            gather_rows(0, 0, gsems[0])

            def piped(g2, _):
                for b in range(3):
                    g = g2 * 3 + b
                    nb = (b + 1) % 3
                    @pl.when(g2 >= 1)
                    def _():
                        pltpu.make_async_copy(
                            wbuf.at[b], den_sh.at[dstP.at[g]],
                            dsems[b]).wait()
                    weights(g, b)
                    pltpu.async_copy(wbuf.at[b], den_sh.at[dstP.at[g]],
                                     dsems[b], add=True)

                    def wait_row_scatter():
                        pltpu.make_async_copy(
                            rowbuf.at[nb], out_sh.at[dstP.at[g]],
                            ssems[nb]).wait()
                    if b < 2:
                        @pl.when(g2 >= 1)
                        def _():
                            wait_row_scatter()
                        gather_rows(g + 1, nb, gsems[nb])
                    else:
                        @pl.when(g2 < CH // 3 - 1)
                        def _():
                            wait_row_scatter()
                            gather_rows(g + 1, nb, gsems[nb])

                    pltpu.make_async_copy(h2d.at[srcP.at[g]],
                                          rowbuf.at[b], gsems[b]).wait()
                    scale_rows(b)
                    pltpu.async_copy(rowbuf.at[b], out_sh.at[dstP.at[g]],
                                     ssems[b], add=True)
                return 0
            lax.fori_loop(0, CH // 3, piped, 0)

            for b in range(3):
                pltpu.make_async_copy(rowbuf.at[b],
                                      out_sh.at[dstP.at[0]],
                                      ssems[b]).wait()
                pltpu.make_async_copy(wbuf.at[b],
                                      den_sh.at[dstP.at[0]],
                                      dsems[b]).wait()

        if bpt == 1:
            sweep_block(wid)
        else:
            for bb in range(bpt):
                sweep_block(bpt * sid + bb)
        plsc.subcore_barrier()

        pltpu.sync_copy(out_sh.at[pl.ds(base, nS)],
                        parts.at[cid, pl.ds(base, nS)])
        pltpu.sync_copy(den_sh.at[pl.ds(base, nS)],
                        dens.at[cid, pl.ds(base, nS)])

    return sc_layer


def kernel(x, edge_index, W1, att_src1, att_dst1, b1,
           W2, att_src2, att_dst2, b2):
    N, Fin = x.shape
    E = edge_index.shape[1]
    H = W1.shape[1]
    C = W2.shape[1]

    NP = (N // (NS * 128) + 1) * (NS * 128)
    Et = E + N
    EP = -(-Et // (NW * 384)) * (NW * 384)
    CH = EP // (NW * 128)
    npad = EP - Et

    ei = edge_index.astype(jnp.int32)
    loops = jnp.arange(N, dtype=jnp.int32)
    pad_src = jnp.arange(npad, dtype=jnp.int32) % N
    pad_dst = N + jnp.arange(npad, dtype=jnp.int32) % (NP - N)
    src = jnp.concatenate([ei[0], loops, pad_src]).reshape(NW, CH, 128)
    dst = jnp.concatenate([ei[1], loops, pad_dst]).reshape(NW, CH, 128)

    xp = jnp.zeros((NP, Fin), jnp.float32).at[:N].set(x)
    A21 = jnp.stack([att_src1, att_dst1], axis=1)
    A22 = jnp.stack([att_src2, att_dst2], axis=1)

    h2d1, as1, ad1 = _tc_prologue(xp, W1, A21)
    sc1 = _make_sc_layer(NP, H // 2, CH, bpt=2, off=NP)
    parts1, dens1 = sc1(h2d1, as1, ad1, src, dst)

    h2, as2, ad2 = _tc_mid(parts1, dens1, b1.reshape(1, H), W2, A22, N)
    sc2 = _make_sc_layer(NP, C, CH, bpt=1, off=0)
    parts2, dens2 = sc2(h2, as2, ad2, src, dst)

    out = _tc_final(parts2, dens2, b2.reshape(1, C))
    return out[:N], edge_index


# speedup vs baseline: 1.5457x; 1.1006x over previous
import functools

import jax
import jax.numpy as jnp
from jax import lax
from jax.experimental import pallas as pl
from jax.experimental.pallas import tpu as pltpu
from jax.experimental.pallas import tpu_sc as plsc

NC, NS, L = 2, 16, 16
NW = NC * NS
NEG = 0.2
BM = 2048


def _prologue_body(x_ref, w_ref, a2_ref, h2_ref, as_ref, ad_ref):
    h = jnp.dot(x_ref[...], w_ref[...], preferred_element_type=jnp.float32)
    hf = h.shape[1] // 2
    h2_ref[0] = h[:, :hf]
    h2_ref[1] = h[:, hf:]
    ap = jnp.dot(h, a2_ref[...], preferred_element_type=jnp.float32)
    as_ref[...] = ap[:, 0]
    ad_ref[...] = ap[:, 1]


def _tc_prologue(xp, W, A2):
    NP, Fin = xp.shape
    H = W.shape[1]
    return pl.pallas_call(
        _prologue_body,
        grid=(NP // BM,),
        in_specs=[pl.BlockSpec((BM, Fin), lambda i: (i, 0)),
                  pl.BlockSpec((Fin, H), lambda i: (0, 0)),
                  pl.BlockSpec((H, 2), lambda i: (0, 0))],
        out_specs=[pl.BlockSpec((2, BM, H // 2), lambda i: (0, i, 0)),
                   pl.BlockSpec((BM,), lambda i: (i,)),
                   pl.BlockSpec((BM,), lambda i: (i,))],
        out_shape=[jax.ShapeDtypeStruct((2, NP, H // 2), jnp.float32),
                   jax.ShapeDtypeStruct((NP,), jnp.float32),
                   jax.ShapeDtypeStruct((NP,), jnp.float32)],
    )(xp, W, A2)


def _make_mid_body(n_valid):
    def _mid_body(p_ref, d_ref, b_ref, w_ref, a2_ref,
                  h_ref, as_ref, ad_ref):
        agg = jnp.concatenate([p_ref[0], p_ref[1]], axis=1)
        rden = (1.0 / (d_ref[0] + 1e-30)).reshape(-1, 1)
        z = jnp.maximum(agg * rden + b_ref[...], 0.0)
        rows = pl.program_id(0) * BM + lax.broadcasted_iota(
            jnp.int32, (BM, 1), 0)
        z = jnp.where(rows < n_valid, z, 0.0)
        h = jnp.dot(z, w_ref[...], preferred_element_type=jnp.float32)
        h_ref[...] = h
        ap = jnp.dot(h, a2_ref[...], preferred_element_type=jnp.float32)
        as_ref[...] = ap[:, 0]
        ad_ref[...] = ap[:, 1]
    return _mid_body


def _tc_mid(parts, dens, b, W, A2, n_valid):
    NP, Fh = parts.shape[1], parts.shape[2]
    H = 2 * Fh
    C = W.shape[1]
    return pl.pallas_call(
        _make_mid_body(n_valid),
        grid=(NP // BM,),
        in_specs=[pl.BlockSpec((2, BM, Fh), lambda i: (0, i, 0)),
                  pl.BlockSpec((2, BM), lambda i: (0, i)),
                  pl.BlockSpec((1, H), lambda i: (0, 0)),
                  pl.BlockSpec((H, C), lambda i: (0, 0)),
                  pl.BlockSpec((C, 2), lambda i: (0, 0))],
        out_specs=[pl.BlockSpec((BM, C), lambda i: (i, 0)),
                   pl.BlockSpec((BM,), lambda i: (i,)),
                   pl.BlockSpec((BM,), lambda i: (i,))],
        out_shape=[jax.ShapeDtypeStruct((NP, C), jnp.float32),
                   jax.ShapeDtypeStruct((NP,), jnp.float32),
                   jax.ShapeDtypeStruct((NP,), jnp.float32)],
    )(parts, dens, b, W, A2)


def _final_body(p_ref, d_ref, b_ref, o_ref):
    rden = (1.0 / (d_ref[0] + d_ref[1] + 1e-30)).reshape(-1, 1)
    o_ref[...] = (p_ref[0] + p_ref[1]) * rden + b_ref[...]


def _tc_final(parts, dens, b):
    NP, C = parts.shape[1], parts.shape[2]
    return pl.pallas_call(
        _final_body,
        grid=(NP // BM,),
        in_specs=[pl.BlockSpec((2, BM, C), lambda i: (0, i, 0)),
                  pl.BlockSpec((2, BM), lambda i: (0, i)),
                  pl.BlockSpec((1, C), lambda i: (0, 0))],
        out_specs=pl.BlockSpec((BM, C), lambda i: (i, 0)),
        out_shape=jax.ShapeDtypeStruct((NP, C), jnp.float32),
    )(parts, dens, b)


def _make_sc_layer(NP, Fh, CH, bpt, off, mb, TR):
    nS = NP // NS
    mesh = plsc.VectorSubcoreMesh(core_axis_name="c", subcore_axis_name="s")

    @functools.partial(
        pl.kernel,
        out_type=[jax.ShapeDtypeStruct((NC, NP, Fh), jnp.float32),
                  jax.ShapeDtypeStruct((NC, NP), jnp.float32)],
        mesh=mesh,
        compiler_params=pltpu.CompilerParams(
            needs_layout_passes=False,
            use_tc_tiling_on_sc=False),
        scratch_types=[
            pltpu.VMEM_SHARED((NP,), jnp.float32),
            pltpu.VMEM_SHARED((NP, Fh), jnp.float32),
            pltpu.VMEM((NP,), jnp.float32),
            pltpu.VMEM((NP,), jnp.float32),
            pltpu.VMEM((CH, 128), jnp.int32),
            pltpu.VMEM((CH, 128), jnp.int32),
            pltpu.VMEM((3, 128), jnp.float32),
            pltpu.VMEM((3, 128, Fh), jnp.float32),
        ] + [pltpu.SemaphoreType.DMA] * 9,
    )
    def sc_layer(h2d, asrc_hbm, adst_hbm, srcM, tailc, parts, dens,
                 den_sh, out_sh, asrc_v, adst_v,
                 srcP, dstP, wbuf, rowbuf,
                 g0, g1, g2, s0, s1, s2, d0, d1, d2):
        gsems = (g0, g1, g2)
        ssems = (s0, s1, s2)
        dsems = (d0, d1, d2)
        cid = lax.axis_index("c")
        sid = lax.axis_index("s")
        wid = cid * NS + sid
        base = sid * nS
        coff = cid * off

        pltpu.sync_copy(asrc_hbm, asrc_v)
        pltpu.sync_copy(adst_hbm, adst_v)

        def zrow(i, _):
            for q in range(Fh // L):
                rowbuf[0, i, pl.ds(q * L, L)] = jnp.zeros((L,), jnp.float32)
            return 0
        lax.fori_loop(0, 128, zrow, 0)
        for q in range(128 // L):
            wbuf[0, pl.ds(q * L, L)] = jnp.zeros((L,), jnp.float32)
        for k in range(nS // 128):
            pltpu.sync_copy(rowbuf.at[0],
                            out_sh.at[pl.ds(base + k * 128, 128)])
            pltpu.sync_copy(wbuf.at[0],
                            den_sh.at[pl.ds(base + k * 128, 128)])
        plsc.subcore_barrier()

        def mx(i, m):
            return jnp.maximum(m, asrc_v[pl.ds(i * L, L)])
        m16 = lax.fori_loop(0, NP // L, mx,
                            jnp.full((L,), -3.0e38, jnp.float32))
        amax = m16[0]
        for i in range(1, L):
            amax = jnp.maximum(amax, m16[i])

        def gather_rows(g, slot, sem):
            pltpu.async_copy(h2d.at[srcP.at[g]], rowbuf.at[slot], sem)

        def weights(g, b):
            for q in range(128 // L):
                sv = srcP[g, pl.ds(q * L, L)] - coff
                dv = dstP[g, pl.ds(q * L, L)]
                a_s = plsc.load_gather(asrc_v, [sv])
                a_d = plsc.load_gather(adst_v, [dv])
                al = a_s + a_d
                al = jnp.maximum(al, NEG * al)
                sh = a_d + amax
                sh = jnp.maximum(sh, NEG * sh)
                wbuf[b, pl.ds(q * L, L)] = jnp.exp(al - sh)

        def scale_rows(b):
            def rgrp(t, _):
                wv = wbuf[b, pl.ds(t * L, L)]
                for rr in range(L):
                    row = t * L + rr
                    ws = wv[rr]
                    for q in range(Fh // L):
                        rowbuf[b, row, pl.ds(q * L, L)] = (
                            rowbuf[b, row, pl.ds(q * L, L)] * ws)
                return 0
            lax.fori_loop(0, 128 // L, rgrp, 0)

        def sweep_block(blk):
            fbB = mb // CH
            mrows = mb - fbB * CH
            @pl.when(blk < fbB)
            def _():
                pltpu.sync_copy(srcM.at[0, pl.ds(blk * CH, CH)], srcP)
                pltpu.sync_copy(srcM.at[1, pl.ds(blk * CH, CH)], dstP)
            if mrows:
                trows = CH - mrows

                @pl.when(blk == fbB)
                def _():
                    pltpu.sync_copy(srcM.at[0, pl.ds(fbB * CH, mrows)],
                                    srcP.at[pl.ds(0, mrows)])
                    pltpu.sync_copy(srcM.at[1, pl.ds(fbB * CH, mrows)],
                                    dstP.at[pl.ds(0, mrows)])
                    pltpu.sync_copy(tailc.at[0, pl.ds(0, trows)],
                                    srcP.at[pl.ds(mrows, trows)])
                    pltpu.sync_copy(tailc.at[1, pl.ds(0, trows)],
                                    dstP.at[pl.ds(mrows, trows)])
            if NW > fbB + 1:
                @pl.when(blk > fbB)
                def _():
                    pltpu.sync_copy(tailc.at[0, pl.ds(blk * CH - mb, CH)],
                                    srcP)
                    pltpu.sync_copy(tailc.at[1, pl.ds(blk * CH - mb, CH)],
                                    dstP)
            if off:
                def addoff(g, _):
                    for q in range(128 // L):
                        srcP[g, pl.ds(q * L, L)] = (
                            srcP[g, pl.ds(q * L, L)] + coff)
                    return 0
                lax.fori_loop(0, CH, addoff, 0)
            gather_rows(0, 0, gsems[0])

            def piped(g2, _):
                for b in range(3):
                    g = g2 * 3 + b
                    nb = (b + 1) % 3
                    @pl.when(g2 >= 1)
                    def _():
                        pltpu.make_async_copy(
                            wbuf.at[b], den_sh.at[dstP.at[g]],
                            dsems[b]).wait()
                    weights(g, b)
                    pltpu.async_copy(wbuf.at[b], den_sh.at[dstP.at[g]],
                                     dsems[b], add=True)

                    def wait_row_scatter():
                        pltpu.make_async_copy(
                            rowbuf.at[nb], out_sh.at[dstP.at[g]],
                            ssems[nb]).wait()
                    if b < 2:
                        @pl.when(g2 >= 1)
                        def _():
                            wait_row_scatter()
                        gather_rows(g + 1, nb, gsems[nb])
                    else:
                        @pl.when(g2 < CH // 3 - 1)
                        def _():
                            wait_row_scatter()
                            gather_rows(g + 1, nb, gsems[nb])

                    pltpu.make_async_copy(h2d.at[srcP.at[g]],
                                          rowbuf.at[b], gsems[b]).wait()
                    scale_rows(b)
                    pltpu.async_copy(rowbuf.at[b], out_sh.at[dstP.at[g]],
                                     ssems[b], add=True)
                return 0
            lax.fori_loop(0, CH // 3, piped, 0)

            for b in range(3):
                pltpu.make_async_copy(rowbuf.at[b],
                                      out_sh.at[dstP.at[0]],
                                      ssems[b]).wait()
                pltpu.make_async_copy(wbuf.at[b],
                                      den_sh.at[dstP.at[0]],
                                      dsems[b]).wait()

        if bpt == 1:
            sweep_block(wid)
        else:
            for bb in range(bpt):
                sweep_block(bpt * sid + bb)
        plsc.subcore_barrier()

        pltpu.sync_copy(out_sh.at[pl.ds(base, nS)],
                        parts.at[cid, pl.ds(base, nS)])
        pltpu.sync_copy(den_sh.at[pl.ds(base, nS)],
                        dens.at[cid, pl.ds(base, nS)])

    return sc_layer


def kernel(x, edge_index, W1, att_src1, att_dst1, b1,
           W2, att_src2, att_dst2, b2):
    N, Fin = x.shape
    E = edge_index.shape[1]
    H = W1.shape[1]
    C = W2.shape[1]

    NP = (N // (NS * 128) + 1) * (NS * 128)
    Et = E + N
    EP = -(-Et // (NW * 384)) * (NW * 384)
    CH = EP // (NW * 128)
    npad = EP - Et

    ei = edge_index.astype(jnp.int32)
    loops = jnp.arange(N, dtype=jnp.int32)
    pad_src = jnp.arange(npad, dtype=jnp.int32) % N
    pad_dst = N + jnp.arange(npad, dtype=jnp.int32) % (NP - N)
    if E % 128 == 0:
        mb = E // 128
        srcM = ei.reshape(2, mb, 128)
        tailc = jnp.stack([jnp.concatenate([loops, pad_src]),
                           jnp.concatenate([loops, pad_dst])])
        tailc = tailc.reshape(2, -1, 128)
    else:
        mb = EP // 128
        srcM = jnp.stack(
            [jnp.concatenate([ei[0], loops, pad_src]),
             jnp.concatenate([ei[1], loops, pad_dst])]).reshape(2, mb, 128)
        tailc = jnp.zeros((2, 1, 128), jnp.int32)
    TR = tailc.shape[1]

    xp = jnp.zeros((NP, Fin), jnp.float32).at[:N].set(x)
    A21 = jnp.stack([att_src1, att_dst1], axis=1)
    A22 = jnp.stack([att_src2, att_dst2], axis=1)

    h2d1, as1, ad1 = _tc_prologue(xp, W1, A21)
    sc1 = _make_sc_layer(NP, H // 2, CH, bpt=2, off=NP, mb=mb, TR=TR)
    parts1, dens1 = sc1(h2d1.reshape(2 * NP, H // 2),
                        as1, ad1, srcM, tailc)

    h2, as2, ad2 = _tc_mid(parts1, dens1, b1.reshape(1, H), W2, A22, N)
    sc2 = _make_sc_layer(NP, C, CH, bpt=1, off=0, mb=mb, TR=TR)
    parts2, dens2 = sc2(h2, as2, ad2, srcM, tailc)

    out = _tc_final(parts2, dens2, b2.reshape(1, C))
    return out[:N], edge_index
